# baseline (speedup 1.0000x reference)
import functools

import jax
import jax.numpy as jnp
from jax import lax
from jax.experimental import pallas as pl
from jax.experimental.pallas import tpu as pltpu
from jax.experimental.pallas import tpu_sc as plsc

L = 16
NS = 16
CHUNK = 128


def _make_sc_loss(T: int, V: int):
    tpw = T // NS
    nchunks = tpw // CHUNK
    ctiles = V // 128
    mesh = plsc.VectorSubcoreMesh(
        core_axis_name="c", subcore_axis_name="s", num_cores=1)

    @functools.partial(
        pl.kernel,
        out_type=jax.ShapeDtypeStruct((L,), jnp.float32),
        mesh=mesh,
        scratch_types=[
            pltpu.VMEM((tpw,), jnp.int32),
            pltpu.VMEM((nchunks, CHUNK), jnp.int32),
            pltpu.VMEM((tpw, 16), jnp.float32),
            pltpu.VMEM((L,), jnp.float32),
            pltpu.VMEM((NS, L), jnp.float32),
            pltpu.VMEM((NS, L), jnp.float32),
            pltpu.VMEM_SHARED((NS, L), jnp.float32),
            pltpu.VMEM_SHARED((NS, L), jnp.float32),
            pltpu.SemaphoreType.DMA,
        ],
        compiler_params=pltpu.CompilerParams(
            use_tc_tiling_on_sc=False, needs_layout_passes=False,
            skip_device_barrier=True),
    )
    def loss_kernel(logp_hbm, tgt_hbm, final_hbm,
                    tgt_v, idx_v, rows_v, stage_v, allsum_v, allcnt_v,
                    shsum_v, shcnt_v, sem):
        w = lax.axis_index("s")
        base = w * tpw
        pltpu.sync_copy(tgt_hbm.at[pl.ds(base, tpw)], tgt_v)

        lanes = lax.iota(jnp.int32, L)
        copies = []
        for j in range(nchunks):
            for i in range(CHUNK // L):
                t = base + j * CHUNK + i * L + lanes
                c = tgt_v[pl.ds(j * CHUNK + i * L, L)]
                krow = ((t >> 3) * ctiles + (c >> 7)) * 8 + (t & 7)
                idx_v[j, pl.ds(i * L, L)] = krow * 8 + ((c >> 4) & 7)
            copies.append(pltpu.async_copy(
                logp_hbm.at[idx_v.at[j]],
                rows_v.at[pl.ds(j * CHUNK, CHUNK)], sem))

        acc = jnp.zeros((L,), jnp.float32)
        cnt = jnp.zeros((L,), jnp.float32)
        for j in range(nchunks):
            copies[j].wait()
            for i in range(CHUNK // L):
                off = j * CHUNK + i * L
                c = tgt_v[pl.ds(off, L)]
                v = plsc.load_gather(rows_v, [off + lanes, c & 15])
                m = c > 1
                acc = acc + jnp.where(m, -v, 0.0)
                cnt = cnt + jnp.where(m, 1.0, 0.0)

        stage_v[...] = acc
        pltpu.sync_copy(stage_v, shsum_v.at[w])
        stage_v[...] = cnt
        pltpu.sync_copy(stage_v, shcnt_v.at[w])
        plsc.subcore_barrier()

        @pl.when(w == 0)
        def _():
            pltpu.sync_copy(shsum_v, allsum_v)
            pltpu.sync_copy(shcnt_v, allcnt_v)
            tot = jnp.zeros((L,), jnp.float32)
            ctot = jnp.zeros((L,), jnp.float32)
            for r in range(NS):
                tot = tot + allsum_v[r, :]
                ctot = ctot + allcnt_v[r, :]

            def lane_total(x):
                for k in (1, 2, 4, 8):
                    x = x + x.at[lanes ^ k].get(mode="promise_in_bounds")
                return x

            stage_v[...] = lane_total(tot) / lane_total(ctot)
            pltpu.sync_copy(stage_v, final_hbm)

    return loss_kernel


def kernel(logp, target):
    B, S, V = logp.shape
    target = target[:, :S]
    T = B * S
    lp_r = (logp.reshape(T // 8, 8, V // 128, 128)
            .transpose(0, 2, 1, 3)
            .reshape(T * V // 16, 16))
    tgt = target.reshape(-1).astype(jnp.int32)
    final = _make_sc_loss(T, V)(lp_r, tgt)
    return final[0]


# speedup vs baseline: 1.0135x; 1.0135x over previous
import functools

import jax
import jax.numpy as jnp
from jax import lax
from jax.experimental import pallas as pl
from jax.experimental.pallas import tpu as pltpu
from jax.experimental.pallas import tpu_sc as plsc

L = 16
NS = 16
CHUNK = 128


def _make_sc_loss(T: int, V: int):
    tpw = T // NS
    nchunks = tpw // CHUNK
    ctiles = V // 128
    mesh = plsc.VectorSubcoreMesh(
        core_axis_name="c", subcore_axis_name="s", num_cores=1)

    @functools.partial(
        pl.kernel,
        out_type=jax.ShapeDtypeStruct((L,), jnp.float32),
        mesh=mesh,
        scratch_types=[
            pltpu.VMEM((nchunks, CHUNK), jnp.int32),
            pltpu.VMEM((nchunks, CHUNK), jnp.int32),
            pltpu.VMEM((tpw, 16), jnp.float32),
            pltpu.VMEM((2, L), jnp.float32),
            pltpu.VMEM((NS, 2, L), jnp.float32),
            pltpu.VMEM_SHARED((NS, 2, L), jnp.float32),
            pltpu.SemaphoreType.DMA,
            pltpu.SemaphoreType.DMA,
        ],
        compiler_params=pltpu.CompilerParams(
            use_tc_tiling_on_sc=False, needs_layout_passes=False,
            skip_device_barrier=True),
    )
    def loss_kernel(logp_hbm, tgt_hbm, final_hbm,
                    tgt_v, idx_v, rows_v, stage_v, allpart_v,
                    shpart_v, sem, tsem):
        w = lax.axis_index("s")
        base = w * tpw
        tcopies = [
            pltpu.async_copy(
                tgt_hbm.at[pl.ds(base + j * CHUNK, CHUNK)], tgt_v.at[j], tsem)
            for j in range(nchunks)
        ]

        lanes = lax.iota(jnp.int32, L)
        copies = []
        for j in range(nchunks):
            tcopies[j].wait()
            for i in range(CHUNK // L):
                t = base + j * CHUNK + i * L + lanes
                c = tgt_v[j, pl.ds(i * L, L)]
                krow = ((t >> 3) * ctiles + (c >> 7)) * 8 + (t & 7)
                idx_v[j, pl.ds(i * L, L)] = krow * 8 + ((c >> 4) & 7)
            copies.append(pltpu.async_copy(
                logp_hbm.at[idx_v.at[j]],
                rows_v.at[pl.ds(j * CHUNK, CHUNK)], sem))

        acc = jnp.zeros((L,), jnp.float32)
        cnt = jnp.zeros((L,), jnp.float32)
        for j in range(nchunks):
            copies[j].wait()
            for i in range(CHUNK // L):
                off = j * CHUNK + i * L
                c = tgt_v[j, pl.ds(i * L, L)]
                v = plsc.load_gather(rows_v, [off + lanes, c & 15])
                m = c > 1
                acc = acc + jnp.where(m, -v, 0.0)
                cnt = cnt + jnp.where(m, 1.0, 0.0)

        stage_v[0, pl.ds(0, L)] = acc
        stage_v[1, pl.ds(0, L)] = cnt
        pltpu.sync_copy(stage_v, shpart_v.at[w])
        plsc.subcore_barrier()

        @pl.when(w == 0)
        def _():
            pltpu.sync_copy(shpart_v, allpart_v)
            tot = jnp.zeros((L,), jnp.float32)
            ctot = jnp.zeros((L,), jnp.float32)
            for r in range(NS):
                tot = tot + allpart_v[r, 0, pl.ds(0, L)]
                ctot = ctot + allpart_v[r, 1, pl.ds(0, L)]

            def lane_total(x):
                for k in (1, 2, 4, 8):
                    x = x + x.at[lanes ^ k].get(mode="promise_in_bounds")
                return x

            stage_v[0, pl.ds(0, L)] = lane_total(tot) / lane_total(ctot)
            pltpu.sync_copy(stage_v.at[0], final_hbm)

    return loss_kernel


def kernel(logp, target):
    B, S, V = logp.shape
    target = target[:, :S]
    T = B * S
    lp_r = (logp.reshape(T // 8, 8, V // 128, 128)
            .transpose(0, 2, 1, 3)
            .reshape(T * V // 16, 16))
    tgt = target.reshape(-1).astype(jnp.int32)
    final = _make_sc_loss(T, V)(lp_r, tgt)
    return final[0]


# speedup vs baseline: 1.0152x; 1.0017x over previous
import functools

import jax
import jax.numpy as jnp
from jax import lax
from jax.experimental import pallas as pl
from jax.experimental.pallas import tpu as pltpu
from jax.experimental.pallas import tpu_sc as plsc

L = 16
NS = 16
CHUNK = 64


def _make_sc_loss(T: int, V: int):
    tpw = T // NS
    nchunks = tpw // CHUNK
    ctiles = V // 128
    mesh = plsc.VectorSubcoreMesh(
        core_axis_name="c", subcore_axis_name="s", num_cores=1)

    @functools.partial(
        pl.kernel,
        out_type=jax.ShapeDtypeStruct((L,), jnp.float32),
        mesh=mesh,
        scratch_types=[
            pltpu.VMEM((nchunks, CHUNK), jnp.int32),
            pltpu.VMEM((nchunks, CHUNK), jnp.int32),
            pltpu.VMEM((tpw, 16), jnp.float32),
            pltpu.VMEM((2, L), jnp.float32),
            pltpu.VMEM((NS, 2, L), jnp.float32),
            pltpu.VMEM_SHARED((NS, 2, L), jnp.float32),
            pltpu.SemaphoreType.DMA,
            pltpu.SemaphoreType.DMA,
        ],
        compiler_params=pltpu.CompilerParams(
            use_tc_tiling_on_sc=False, needs_layout_passes=False,
            skip_device_barrier=True),
    )
    def loss_kernel(logp_hbm, tgt_hbm, final_hbm,
                    tgt_v, idx_v, rows_v, stage_v, allpart_v,
                    shpart_v, sem, tsem):
        w = lax.axis_index("s")
        base = w * tpw
        tcopies = [
            pltpu.async_copy(
                tgt_hbm.at[pl.ds(base + j * CHUNK, CHUNK)], tgt_v.at[j], tsem)
            for j in range(nchunks)
        ]

        lanes = lax.iota(jnp.int32, L)
        copies = []
        for j in range(nchunks):
            tcopies[j].wait()
            for i in range(CHUNK // L):
                t = base + j * CHUNK + i * L + lanes
                c = tgt_v[j, pl.ds(i * L, L)]
                krow = ((t >> 3) * ctiles + (c >> 7)) * 8 + (t & 7)
                idx_v[j, pl.ds(i * L, L)] = krow * 8 + ((c >> 4) & 7)
            copies.append(pltpu.async_copy(
                logp_hbm.at[idx_v.at[j]],
                rows_v.at[pl.ds(j * CHUNK, CHUNK)], sem))

        acc = jnp.zeros((L,), jnp.float32)
        cnt = jnp.zeros((L,), jnp.float32)
        for j in range(nchunks):
            copies[j].wait()
            for i in range(CHUNK // L):
                off = j * CHUNK + i * L
                c = tgt_v[j, pl.ds(i * L, L)]
                v = plsc.load_gather(rows_v, [off + lanes, c & 15])
                m = c > 1
                acc = acc + jnp.where(m, -v, 0.0)
                cnt = cnt + jnp.where(m, 1.0, 0.0)

        stage_v[0, pl.ds(0, L)] = acc
        stage_v[1, pl.ds(0, L)] = cnt
        pltpu.sync_copy(stage_v, shpart_v.at[w])
        plsc.subcore_barrier()

        @pl.when(w == 0)
        def _():
            pltpu.sync_copy(shpart_v, allpart_v)
            tot = jnp.zeros((L,), jnp.float32)
            ctot = jnp.zeros((L,), jnp.float32)
            for r in range(NS):
                tot = tot + allpart_v[r, 0, pl.ds(0, L)]
                ctot = ctot + allpart_v[r, 1, pl.ds(0, L)]

            def lane_total(x):
                for k in (1, 2, 4, 8):
                    x = x + x.at[lanes ^ k].get(mode="promise_in_bounds")
                return x

            stage_v[0, pl.ds(0, L)] = lane_total(tot) / lane_total(ctot)
            pltpu.sync_copy(stage_v.at[0], final_hbm)

    return loss_kernel


def kernel(logp, target):
    B, S, V = logp.shape
    target = target[:, :S]
    T = B * S
    lp_r = (logp.reshape(T // 8, 8, V // 128, 128)
            .transpose(0, 2, 1, 3)
            .reshape(T * V // 16, 16))
    tgt = target.reshape(-1).astype(jnp.int32)
    final = _make_sc_loss(T, V)(lp_r, tgt)
    return final[0]
